# Initial kernel scaffold; baseline (speedup 1.0000x reference)
import jax
import jax.numpy as jnp
from jax.experimental import pallas as pl


def kernel(vision_feats, text_tokens, embed_table, vision_xattn_mask, buffer_xattn_mask):
    raise NotImplementedError("write your pallas kernel here")


# speedup vs baseline: 1.1821x; 1.1821x over previous
import functools

import jax
import jax.numpy as jnp
from jax import lax
from jax.experimental import pallas as pl
from jax.experimental.pallas import tpu as pltpu
from jax.experimental.pallas import tpu_sc as plsc

_D = 1024
_B = 4 * 4096
_NC = 2
_NS = 16
_NW = _NC * _NS
_BPW = _B // _NW
_CH = 32
_NCHUNK = _BPW // _CH


@functools.partial(
    pl.kernel,
    out_type=jax.ShapeDtypeStruct((_B, _D), jnp.float32),
    mesh=plsc.VectorSubcoreMesh(
        core_axis_name="c", subcore_axis_name="s",
        num_cores=_NC, num_subcores=_NS,
    ),
    scratch_types=[
        pltpu.VMEM((_BPW,), jnp.int32),
        pltpu.VMEM((2, _CH, _D), jnp.float32),
        pltpu.SemaphoreType.DMA,
        pltpu.SemaphoreType.DMA,
        pltpu.SemaphoreType.DMA,
        pltpu.SemaphoreType.DMA,
    ],
)
def _embed_gather(table_hbm, idx_hbm, out_hbm, idx_v, bufs,
                  gsem0, gsem1, ssem0, ssem1):
    wid = lax.axis_index("s") * _NC + lax.axis_index("c")
    base = wid * _BPW
    pltpu.sync_copy(idx_hbm.at[pl.ds(base, _BPW)], idx_v)

    gsems = (gsem0, gsem1)
    ssems = (ssem0, ssem1)
    gathers = [None, None]
    scatters = [None, None]

    def start_gather(c):
        b = c % 2
        gathers[b] = pltpu.async_copy(
            table_hbm.at[idx_v.at[pl.ds(c * _CH, _CH)]],
            bufs.at[b],
            gsems[b],
        )

    start_gather(0)
    for c in range(_NCHUNK):
        b = c % 2
        if c + 1 < _NCHUNK:
            if scatters[1 - b] is not None:
                scatters[1 - b].wait()
            start_gather(c + 1)
        gathers[b].wait()
        scatters[b] = pltpu.async_copy(
            bufs.at[b],
            out_hbm.at[pl.ds(base + c * _CH, _CH)],
            ssems[b],
        )
    scatters[0].wait()
    scatters[1].wait()


def kernel(vision_feats, text_tokens, embed_table,
           vision_xattn_mask, buffer_xattn_mask):
    idx = text_tokens.reshape(-1)
    emb = _embed_gather(embed_table, idx)
    embedded_text = emb.reshape(
        text_tokens.shape[0], text_tokens.shape[1], _D)
    return (
        embedded_text,
        vision_feats,
        vision_feats,
        vision_xattn_mask,
        buffer_xattn_mask,
    )
